# baseline (speedup 1.0000x reference)
import jax
import jax.numpy as jnp
from jax.experimental import pallas as pl
from jax.experimental.pallas import tpu as pltpu

_S = 2


def _unpool_body(x_ref, s_ref, r_ref, o_ref):
    n, h, w = x_ref.shape
    x2 = x_ref[...].reshape(n * h, w)
    y = jnp.dot(x2, s_ref[...], preferred_element_type=jnp.float32,
                precision=jax.lax.Precision.HIGHEST)
    y = y.reshape(n, h, _S * w)
    for k in range(n):
        o_ref[k] = jnp.dot(r_ref[...], y[k], preferred_element_type=jnp.float32,
                           precision=jax.lax.Precision.HIGHEST)


@jax.jit
def kernel(x):
    b, c, h, w = x.shape
    bc = b * c
    n_img = 8
    grid = (bc // n_img,)

    s_rows = jax.lax.broadcasted_iota(jnp.int32, (w, _S * w), 0)
    s_cols = jax.lax.broadcasted_iota(jnp.int32, (w, _S * w), 1)
    s = (s_cols == _S * s_rows).astype(jnp.float32)
    r_rows = jax.lax.broadcasted_iota(jnp.int32, (_S * h, h), 0)
    r_cols = jax.lax.broadcasted_iota(jnp.int32, (_S * h, h), 1)
    r = (r_rows == _S * r_cols).astype(jnp.float32)

    xf = x.reshape(bc, h, w)
    out = pl.pallas_call(
        _unpool_body,
        grid=grid,
        in_specs=[
            pl.BlockSpec((n_img, h, w), lambda i: (i, 0, 0)),
            pl.BlockSpec((w, _S * w), lambda i: (0, 0)),
            pl.BlockSpec((_S * h, h), lambda i: (0, 0)),
        ],
        out_specs=pl.BlockSpec((n_img, _S * h, _S * w), lambda i: (i, 0, 0)),
        out_shape=jax.ShapeDtypeStruct((bc, _S * h, _S * w), x.dtype),
        compiler_params=pltpu.CompilerParams(
            dimension_semantics=("parallel",),
        ),
    )(xf, s, r)
    return out.reshape(b, c, _S * h, _S * w)


# speedup vs baseline: 1.0007x; 1.0007x over previous
import jax
import jax.numpy as jnp
from jax.experimental import pallas as pl
from jax.experimental.pallas import tpu as pltpu

_S = 2


def _unpool_body(x_ref, o_ref):
    n, h, w = x_ref.shape
    x = x_ref[...]
    lane = jax.lax.broadcasted_iota(jnp.int32, (n, h, w), 2)
    even = lane % _S == 0
    y0 = jnp.where(even, jnp.take_along_axis(x, lane // _S, axis=2), 0.0)
    y1 = jnp.where(even, jnp.take_along_axis(x, w // _S + lane // _S, axis=2), 0.0)
    o_ref[:, :: _S, 0:1, :] = y0[:, :, None, :]
    o_ref[:, :: _S, 1:2, :] = y1[:, :, None, :]
    o_ref[:, 1 :: _S, 0:1, :] = jnp.zeros_like(y0)[:, :, None, :]
    o_ref[:, 1 :: _S, 1:2, :] = jnp.zeros_like(y0)[:, :, None, :]


@jax.jit
def kernel(x):
    b, c, h, w = x.shape
    bc = b * c
    n_img = 8
    grid = (bc // n_img,)

    xf = x.reshape(bc, h, w)
    out = pl.pallas_call(
        _unpool_body,
        grid=grid,
        in_specs=[pl.BlockSpec((n_img, h, w), lambda i: (i, 0, 0))],
        out_specs=pl.BlockSpec((n_img, _S * h, _S, w), lambda i: (i, 0, 0, 0)),
        out_shape=jax.ShapeDtypeStruct((bc, _S * h, _S, w), x.dtype),
        compiler_params=pltpu.CompilerParams(
            dimension_semantics=("parallel",),
        ),
    )(xf)
    return out.reshape(b, c, _S * h, _S * w)


# speedup vs baseline: 2.6599x; 2.6580x over previous
import jax
import jax.numpy as jnp
from jax.experimental import pallas as pl
from jax.experimental.pallas import tpu as pltpu

_S = 2


def _unpool_body(x_ref, o_ref):
    n, h, w = x_ref.shape
    x = x_ref[...]
    lane = jax.lax.broadcasted_iota(jnp.int32, (n, h, _S * w), 2)
    rep = jnp.take_along_axis(x, lane // _S, axis=2)
    y = jnp.where(lane % _S == 0, rep, 0.0)
    z = jnp.stack([y, jnp.zeros_like(y)], axis=2)
    o_ref[...] = z.reshape(n, _S * h, _S * w)


@jax.jit
def kernel(x):
    b, c, h, w = x.shape
    bc = b * c
    n_img = 8
    grid = (bc // n_img,)

    xf = x.reshape(bc, h, w)
    out = pl.pallas_call(
        _unpool_body,
        grid=grid,
        in_specs=[pl.BlockSpec((n_img, h, w), lambda i: (i, 0, 0))],
        out_specs=pl.BlockSpec((n_img, _S * h, _S * w), lambda i: (i, 0, 0)),
        out_shape=jax.ShapeDtypeStruct((bc, _S * h, _S * w), x.dtype),
        compiler_params=pltpu.CompilerParams(
            dimension_semantics=("parallel",),
        ),
    )(xf)
    return out.reshape(b, c, _S * h, _S * w)


# speedup vs baseline: 2.9484x; 1.1085x over previous
import jax
import jax.numpy as jnp
from jax.experimental import pallas as pl
from jax.experimental.pallas import tpu as pltpu

_S = 2


def _unpool_body(x_ref, o_ref):
    n, h, w = x_ref.shape
    x = x_ref[...]
    lane = jax.lax.broadcasted_iota(jnp.int32, (n, h, _S * w), 2)
    rep = jnp.take_along_axis(x, lane // _S, axis=2)
    y = jnp.where(lane % _S == 0, rep, 0.0)
    z = jnp.stack([y, jnp.zeros_like(y)], axis=2)
    o_ref[...] = z.reshape(n, _S * h, _S * w)


@jax.jit
def kernel(x):
    b, c, h, w = x.shape
    bc = b * c
    n_img = 16
    grid = (bc // n_img,)

    xf = x.reshape(bc, h, w)
    out = pl.pallas_call(
        _unpool_body,
        grid=grid,
        in_specs=[pl.BlockSpec((n_img, h, w), lambda i: (i, 0, 0))],
        out_specs=pl.BlockSpec((n_img, _S * h, _S * w), lambda i: (i, 0, 0)),
        out_shape=jax.ShapeDtypeStruct((bc, _S * h, _S * w), x.dtype),
        compiler_params=pltpu.CompilerParams(
            dimension_semantics=("parallel",),
        ),
    )(xf)
    return out.reshape(b, c, _S * h, _S * w)


# speedup vs baseline: 2.9884x; 1.0136x over previous
import jax
import jax.numpy as jnp
from jax.experimental import pallas as pl
from jax.experimental.pallas import tpu as pltpu

_S = 2


def _unpool_body(x_ref, o_ref):
    n, h, w = x_ref.shape
    x = x_ref[...]
    lane = jax.lax.broadcasted_iota(jnp.int32, (n, h, _S * w), 2)
    rep = jnp.take_along_axis(x, lane // _S, axis=2)
    y = jnp.where(lane % _S == 0, rep, 0.0)
    z = jnp.stack([y, jnp.zeros_like(y)], axis=2)
    o_ref[...] = z.reshape(n, _S * h, _S * w)


@jax.jit
def kernel(x):
    b, c, h, w = x.shape
    bc = b * c
    n_img = 32
    grid = (bc // n_img,)

    xf = x.reshape(bc, h, w)
    out = pl.pallas_call(
        _unpool_body,
        grid=grid,
        in_specs=[pl.BlockSpec((n_img, h, w), lambda i: (i, 0, 0))],
        out_specs=pl.BlockSpec((n_img, _S * h, _S * w), lambda i: (i, 0, 0)),
        out_shape=jax.ShapeDtypeStruct((bc, _S * h, _S * w), x.dtype),
        compiler_params=pltpu.CompilerParams(
            dimension_semantics=("parallel",),
        ),
    )(xf)
    return out.reshape(b, c, _S * h, _S * w)


# speedup vs baseline: 2.9892x; 1.0003x over previous
import jax
import jax.numpy as jnp
from jax.experimental import pallas as pl
from jax.experimental.pallas import tpu as pltpu

_S = 2


def _unpool_body(x_ref, o_ref):
    n, h, w = x_ref.shape
    x = x_ref[...]
    lane = jax.lax.broadcasted_iota(jnp.int32, (n, h, _S * w), 2)
    rep = jnp.take_along_axis(x, lane // _S, axis=2)
    y = jnp.where(lane % _S == 0, rep, 0.0)
    z = jnp.stack([y, jnp.zeros_like(y)], axis=2)
    o_ref[...] = z.reshape(n, _S * h, _S * w)


@jax.jit
def kernel(x):
    b, c, h, w = x.shape
    bc = b * c
    n_img = 64
    grid = (bc // n_img,)

    xf = x.reshape(bc, h, w)
    out = pl.pallas_call(
        _unpool_body,
        grid=grid,
        in_specs=[pl.BlockSpec((n_img, h, w), lambda i: (i, 0, 0))],
        out_specs=pl.BlockSpec((n_img, _S * h, _S * w), lambda i: (i, 0, 0)),
        out_shape=jax.ShapeDtypeStruct((bc, _S * h, _S * w), x.dtype),
        compiler_params=pltpu.CompilerParams(
            dimension_semantics=("parallel",),
            vmem_limit_bytes=50 * 1024 * 1024,
        ),
    )(xf)
    return out.reshape(b, c, _S * h, _S * w)
